# baseline (speedup 1.0000x reference)
import jax
import jax.numpy as jnp
from jax import lax
from jax.experimental import pallas as pl
from jax.experimental.pallas import tpu as pltpu
from jax.experimental.pallas import tpu_sc as plsc

B = 16384
L = 50
D = 64
N = B * L

NW = 32
BB_PER_W = (B // 128) // NW
ITEMS = L * BB_PER_W


def kernel(indices, table):
    tablep = jnp.pad(table, ((0, 0), (0, 128 - D)))
    idx_t = indices.T.astype(jnp.int32)

    mesh = plsc.VectorSubcoreMesh(core_axis_name="core", subcore_axis_name="subcore")

    @pl.kernel(
        out_type=jax.ShapeDtypeStruct((L, 8, 128, 8, 128), table.dtype),
        mesh=mesh,
        scratch_types=[
            pltpu.VMEM((L, 128 * BB_PER_W), jnp.int32),
            pltpu.VMEM((2, 128, 128), jnp.float32),
            pltpu.VMEM((2, 8, 8, 128), jnp.float32),
            pltpu.SemaphoreType.DMA,
            pltpu.SemaphoreType.DMA,
            pltpu.SemaphoreType.DMA,
        ],
        compiler_params=pltpu.CompilerParams(
            use_tc_tiling_on_sc=False, needs_layout_passes=False
        ),
    )
    def gather_kernel(tab_hbm, idx_hbm, out_hbm, idxall, rows, tbuf, isem, gsem, wsem):
        wid = lax.axis_index("subcore") * 2 + lax.axis_index("core")
        base_b = wid * (128 * BB_PER_W)
        pltpu.async_copy(idx_hbm.at[:, pl.ds(base_b, 128 * BB_PER_W)], idxall, isem).wait()

        iota = lax.iota(jnp.int32, 16)

        def fire_gather(t, s):
            l = t // BB_PER_W
            j = t % BB_PER_W
            return pltpu.async_copy(
                tab_hbm.at[idxall.at[l, pl.ds(j * 128, 128)]], rows.at[s], gsem
            )

        def transpose(t, s):
            @pl.loop(0, 8)
            def _(dbk):
                for ds in range(8):
                    d_vec = jnp.full((16,), 0, jnp.int32) + (dbk * 8 + ds)
                    for k in range(8):
                        v = plsc.load_gather(rows.at[s], [iota + k * 16, d_vec])
                        tbuf[s, dbk, ds, pl.ds(k * 16, 16)] = v

        def fire_write(t, s):
            l = t // BB_PER_W
            j = t % BB_PER_W
            return pltpu.async_copy(
                tbuf.at[s], out_hbm.at[l, :, wid * BB_PER_W + j], wsem
            )

        @pl.loop(0, ITEMS, step=2)
        def _(g):
            da = fire_gather(g, 0)
            db_ = fire_gather(g + 1, 1)
            da.wait()
            transpose(g, 0)
            wa = fire_write(g, 0)
            db_.wait()
            transpose(g + 1, 1)
            wb = fire_write(g + 1, 1)
            wa.wait()
            wb.wait()

    out5 = gather_kernel(tablep, idx_t)
    return out5.transpose(2, 4, 0, 1, 3).reshape(B, L, D)


# speedup vs baseline: 1.3709x; 1.3709x over previous
import jax
import jax.numpy as jnp
from jax import lax
from jax.experimental import pallas as pl
from jax.experimental.pallas import tpu as pltpu
from jax.experimental.pallas import tpu_sc as plsc

B = 16384
L = 50
D = 64
N = B * L

NW = 32
BB_PER_W = (B // 128) // NW
ITEMS = L * BB_PER_W


def kernel(indices, table):
    tablep = jnp.pad(table, ((0, 0), (0, 128 - D)))
    idx_t = indices.T.astype(jnp.int32)

    mesh = plsc.VectorSubcoreMesh(core_axis_name="core", subcore_axis_name="subcore")

    @pl.kernel(
        out_type=jax.ShapeDtypeStruct((L, 8, 128, 8, 128), table.dtype),
        mesh=mesh,
        scratch_types=[
            pltpu.VMEM((L, 128 * BB_PER_W), jnp.int32),
            pltpu.VMEM((2, 128, 128), jnp.float32),
            pltpu.VMEM((2, 8, 8, 128), jnp.float32),
            pltpu.SemaphoreType.DMA,
            pltpu.SemaphoreType.DMA,
            pltpu.SemaphoreType.DMA,
        ],
        compiler_params=pltpu.CompilerParams(
            use_tc_tiling_on_sc=False, needs_layout_passes=False
        ),
    )
    def gather_kernel(tab_hbm, idx_hbm, out_hbm, idxall, rows, tbuf, isem, gsem, wsem):
        wid = lax.axis_index("subcore") * 2 + lax.axis_index("core")
        base_b = wid * (128 * BB_PER_W)
        pltpu.async_copy(idx_hbm.at[:, pl.ds(base_b, 128 * BB_PER_W)], idxall, isem).wait()

        iota = lax.iota(jnp.int32, 16)
        ds_vec = lax.rem(iota, 8)
        db_vecs = [iota // 8 + 2 * k for k in range(4)]

        def fire_gather(t, s):
            l = t // BB_PER_W
            j = t % BB_PER_W
            return pltpu.async_copy(
                tab_hbm.at[idxall.at[l, pl.ds(j * 128, 128)]], rows.at[s], gsem
            )

        def transpose(t, s):
            @plsc.parallel_loop(0, 128, unroll=4)
            def _(bs):
                bs_vec = jnp.full((16,), 0, jnp.int32) + bs
                for k in range(4):
                    v = rows[s, bs, pl.ds(16 * k, 16)]
                    plsc.store_scatter(tbuf.at[s], [db_vecs[k], ds_vec, bs_vec], v)

        def fire_write(t, s):
            l = t // BB_PER_W
            j = t % BB_PER_W
            return pltpu.async_copy(
                tbuf.at[s], out_hbm.at[l, :, wid * BB_PER_W + j], wsem
            )

        @pl.loop(0, ITEMS, step=2)
        def _(g):
            da = fire_gather(g, 0)
            db_ = fire_gather(g + 1, 1)
            da.wait()
            transpose(g, 0)
            wa = fire_write(g, 0)
            db_.wait()
            transpose(g + 1, 1)
            wb = fire_write(g + 1, 1)
            wa.wait()
            wb.wait()

    out5 = gather_kernel(tablep, idx_t)
    return out5.transpose(2, 4, 0, 1, 3).reshape(B, L, D)


# speedup vs baseline: 1.3724x; 1.0011x over previous
import jax
import jax.numpy as jnp
from jax import lax
from jax.experimental import pallas as pl
from jax.experimental.pallas import tpu as pltpu
from jax.experimental.pallas import tpu_sc as plsc

B = 16384
L = 50
D = 64
N = B * L

NW = 32
BB_PER_W = (B // 128) // NW
ITEMS = L * BB_PER_W


def kernel(indices, table):
    tablep = jnp.pad(table, ((0, 0), (0, 128 - D)))
    idx_t = indices.T.astype(jnp.int32)

    mesh = plsc.VectorSubcoreMesh(core_axis_name="core", subcore_axis_name="subcore")

    @pl.kernel(
        out_type=jax.ShapeDtypeStruct((L, 8, 128, 8, 128), table.dtype),
        mesh=mesh,
        scratch_types=[
            pltpu.VMEM((L, 128 * BB_PER_W), jnp.int32),
            pltpu.VMEM((2, 128, 128), jnp.float32),
            pltpu.VMEM((2, 8, 8, 128), jnp.float32),
            pltpu.SemaphoreType.DMA,
            pltpu.SemaphoreType.DMA,
            pltpu.SemaphoreType.DMA,
        ],
        compiler_params=pltpu.CompilerParams(
            use_tc_tiling_on_sc=False, needs_layout_passes=False
        ),
    )
    def gather_kernel(tab_hbm, idx_hbm, out_hbm, idxall, rows, tbuf, isem, gsem, wsem):
        wid = lax.axis_index("subcore") * 2 + lax.axis_index("core")
        base_b = wid * (128 * BB_PER_W)
        pltpu.async_copy(idx_hbm.at[:, pl.ds(base_b, 128 * BB_PER_W)], idxall, isem).wait()

        iota = lax.iota(jnp.int32, 16)
        ds_vec = lax.rem(iota, 8)
        db_vecs = [iota // 8 + 2 * k for k in range(4)]

        def fire_gather(t, s):
            l = t // BB_PER_W
            j = t % BB_PER_W
            return pltpu.async_copy(
                tab_hbm.at[idxall.at[l, pl.ds(j * 128, 128)]], rows.at[s], gsem
            )

        def transpose(t, s):
            @plsc.parallel_loop(0, 128, unroll=8)
            def _(bs):
                bs_vec = jnp.full((16,), 0, jnp.int32) + bs
                for k in range(4):
                    v = rows[s, bs, pl.ds(16 * k, 16)]
                    plsc.store_scatter(tbuf.at[s], [db_vecs[k], ds_vec, bs_vec], v)

        def fire_write(t, s):
            l = t // BB_PER_W
            j = t % BB_PER_W
            return pltpu.async_copy(
                tbuf.at[s], out_hbm.at[l, :, wid * BB_PER_W + j], wsem
            )

        @pl.loop(0, ITEMS, step=2)
        def _(g):
            da = fire_gather(g, 0)
            db_ = fire_gather(g + 1, 1)
            da.wait()
            transpose(g, 0)
            wa = fire_write(g, 0)
            db_.wait()
            transpose(g + 1, 1)
            wb = fire_write(g + 1, 1)
            wa.wait()
            wb.wait()

    out5 = gather_kernel(tablep, idx_t)
    return out5.transpose(2, 4, 0, 1, 3).reshape(B, L, D)


# speedup vs baseline: 1.4071x; 1.0253x over previous
import jax
import jax.numpy as jnp
from jax import lax
from jax.experimental import pallas as pl
from jax.experimental.pallas import tpu as pltpu
from jax.experimental.pallas import tpu_sc as plsc

B = 16384
L = 50
D = 64
N = B * L
VOCAB = 1000000

NW = 32
BB_PER_W = (B // 128) // NW
ITEMS = L * BB_PER_W


def kernel(indices, table):
    tablep = jnp.pad(table, ((0, 0), (0, 128 - D))).reshape(2 * VOCAB, D)
    idx_t = indices.T.astype(jnp.int32) * 2

    mesh = plsc.VectorSubcoreMesh(core_axis_name="core", subcore_axis_name="subcore")

    @pl.kernel(
        out_type=jax.ShapeDtypeStruct((L, 8, 128, 8, 128), table.dtype),
        mesh=mesh,
        scratch_types=[
            pltpu.VMEM((L, 128 * BB_PER_W), jnp.int32),
            pltpu.VMEM((2, 128, D), jnp.float32),
            pltpu.VMEM((2, 8, 8, 128), jnp.float32),
            pltpu.SemaphoreType.DMA,
            pltpu.SemaphoreType.DMA,
            pltpu.SemaphoreType.DMA,
        ],
        compiler_params=pltpu.CompilerParams(
            use_tc_tiling_on_sc=False, needs_layout_passes=False
        ),
    )
    def gather_kernel(tab_hbm, idx_hbm, out_hbm, idxall, rows, tbuf, isem, gsem, wsem):
        wid = lax.axis_index("subcore") * 2 + lax.axis_index("core")
        base_b = wid * (128 * BB_PER_W)
        pltpu.async_copy(idx_hbm.at[:, pl.ds(base_b, 128 * BB_PER_W)], idxall, isem).wait()

        iota = lax.iota(jnp.int32, 16)
        ds_vec = lax.rem(iota, 8)
        db_vecs = [iota // 8 + 2 * k for k in range(4)]

        def fire_gather(t, s):
            l = t // BB_PER_W
            j = t % BB_PER_W
            return pltpu.async_copy(
                tab_hbm.at[idxall.at[l, pl.ds(j * 128, 128)]], rows.at[s], gsem
            )

        def transpose(t, s):
            @plsc.parallel_loop(0, 128, unroll=8)
            def _(bs):
                bs_vec = jnp.full((16,), 0, jnp.int32) + bs
                for k in range(4):
                    v = rows[s, bs, pl.ds(16 * k, 16)]
                    plsc.store_scatter(tbuf.at[s], [db_vecs[k], ds_vec, bs_vec], v)

        def fire_write(t, s):
            l = t // BB_PER_W
            j = t % BB_PER_W
            return pltpu.async_copy(
                tbuf.at[s], out_hbm.at[l, :, wid * BB_PER_W + j], wsem
            )

        @pl.loop(0, ITEMS, step=2)
        def _(g):
            da = fire_gather(g, 0)
            db_ = fire_gather(g + 1, 1)
            da.wait()
            transpose(g, 0)
            wa = fire_write(g, 0)
            db_.wait()
            transpose(g + 1, 1)
            wb = fire_write(g + 1, 1)
            wa.wait()
            wb.wait()

    out5 = gather_kernel(tablep, idx_t)
    return out5.transpose(2, 4, 0, 1, 3).reshape(B, L, D)


# speedup vs baseline: 1.4804x; 1.0521x over previous
import jax
import jax.numpy as jnp
from jax import lax
from jax.experimental import pallas as pl
from jax.experimental.pallas import tpu as pltpu
from jax.experimental.pallas import tpu_sc as plsc

B = 16384
L = 50
D = 64
N = B * L
VOCAB = 1000000

NW = 32
BB_PER_W = (B // 128) // NW
ITEMS = L * BB_PER_W


def kernel(indices, table):
    tablep = jnp.pad(table, ((0, 0), (0, 128 - D))).reshape(2 * VOCAB, D)
    idx_t = indices.T.astype(jnp.int32) * 2

    mesh = plsc.VectorSubcoreMesh(core_axis_name="core", subcore_axis_name="subcore")

    @pl.kernel(
        out_type=jax.ShapeDtypeStruct((L, 8, 128, 8, 128), table.dtype),
        mesh=mesh,
        scratch_types=[
            pltpu.VMEM((L, 128 * BB_PER_W), jnp.int32),
            pltpu.VMEM((2, 128 * BB_PER_W, D), jnp.float32),
            pltpu.VMEM((2, 8, 8, 128), jnp.float32),
            pltpu.SemaphoreType.DMA,
            pltpu.SemaphoreType.DMA,
            pltpu.SemaphoreType.DMA,
        ],
        compiler_params=pltpu.CompilerParams(
            use_tc_tiling_on_sc=False, needs_layout_passes=False
        ),
    )
    def gather_kernel(tab_hbm, idx_hbm, out_hbm, idxall, rows, tbuf, isem, gsem, wsem):
        wid = lax.axis_index("subcore") * 2 + lax.axis_index("core")
        base_b = wid * (128 * BB_PER_W)
        pltpu.async_copy(idx_hbm.at[:, pl.ds(base_b, 128 * BB_PER_W)], idxall, isem).wait()

        iota = lax.iota(jnp.int32, 16)
        ds_vec = lax.rem(iota, 8)
        db_vecs = [iota // 8 + 2 * k for k in range(4)]

        def fire_gather(l, s):
            return pltpu.async_copy(tab_hbm.at[idxall.at[l]], rows.at[s], gsem)

        def transpose(s, j, tb):
            @plsc.parallel_loop(0, 128, unroll=8)
            def _(bs):
                bs_vec = jnp.full((16,), 0, jnp.int32) + bs
                for k in range(4):
                    v = rows[s, j * 128 + bs, pl.ds(16 * k, 16)]
                    plsc.store_scatter(tbuf.at[tb], [db_vecs[k], ds_vec, bs_vec], v)

        def fire_write(l, j, tb):
            return pltpu.async_copy(
                tbuf.at[tb], out_hbm.at[l, :, wid * BB_PER_W + j], wsem
            )

        def process(l, s):
            transpose(s, 0, 0)
            w0 = fire_write(l, 0, 0)
            transpose(s, 1, 1)
            w1 = fire_write(l, 1, 1)
            w0.wait()
            transpose(s, 2, 0)
            w2 = fire_write(l, 2, 0)
            w1.wait()
            transpose(s, 3, 1)
            w3 = fire_write(l, 3, 1)
            w2.wait()
            w3.wait()

        @pl.loop(0, L, step=2)
        def _(g):
            da = fire_gather(g, 0)
            db_ = fire_gather(g + 1, 1)
            da.wait()
            process(g, 0)
            db_.wait()
            process(g + 1, 1)

    out5 = gather_kernel(tablep, idx_t)
    return out5.transpose(2, 4, 0, 1, 3).reshape(B, L, D)


# speedup vs baseline: 1.5398x; 1.0401x over previous
import jax
import jax.numpy as jnp
from jax import lax
from jax.experimental import pallas as pl
from jax.experimental.pallas import tpu as pltpu
from jax.experimental.pallas import tpu_sc as plsc

B = 16384
L = 50
D = 64
N = B * L
VOCAB = 1000000

NW = 32
BB_PER_W = (B // 128) // NW
ITEMS = L * BB_PER_W


def kernel(indices, table):
    tablep = jnp.pad(table, ((0, 0), (0, 128 - D))).reshape(2 * VOCAB, D)
    idx_t = indices.T.astype(jnp.int32) * 2

    mesh = plsc.VectorSubcoreMesh(core_axis_name="core", subcore_axis_name="subcore")

    @pl.kernel(
        out_type=jax.ShapeDtypeStruct((L, 8, 128, 8, 128), table.dtype),
        mesh=mesh,
        scratch_types=[
            pltpu.VMEM((L, 128 * BB_PER_W), jnp.int32),
            pltpu.VMEM((2, 128 * BB_PER_W, D), jnp.float32),
            pltpu.VMEM((2, 8, 8, 128), jnp.float32),
            pltpu.SemaphoreType.DMA,
            pltpu.SemaphoreType.DMA,
            pltpu.SemaphoreType.DMA,
        ],
        compiler_params=pltpu.CompilerParams(
            use_tc_tiling_on_sc=False, needs_layout_passes=False
        ),
    )
    def gather_kernel(tab_hbm, idx_hbm, out_hbm, idxall, rows, tbuf, isem, gsem, wsem):
        wid = lax.axis_index("subcore") * 2 + lax.axis_index("core")
        base_b = wid * (128 * BB_PER_W)
        pltpu.async_copy(idx_hbm.at[:, pl.ds(base_b, 128 * BB_PER_W)], idxall, isem).wait()

        iota = lax.iota(jnp.int32, 16)
        ds_vec = lax.rem(iota, 8)
        db_vecs = [iota // 8 + 2 * k for k in range(4)]

        def fire_gather(l, s):
            return pltpu.async_copy(tab_hbm.at[idxall.at[l]], rows.at[s], gsem)

        def transpose(s, j, tb):
            @plsc.parallel_loop(0, 128, unroll=8)
            def _(bs):
                bs_vec = jnp.full((16,), 0, jnp.int32) + bs
                for k in range(4):
                    v = rows[s, j * 128 + bs, pl.ds(16 * k, 16)]
                    plsc.store_scatter(tbuf.at[tb], [db_vecs[k], ds_vec, bs_vec], v)

        def fire_write(l, j, tb):
            return pltpu.async_copy(
                tbuf.at[tb], out_hbm.at[l, :, wid * BB_PER_W + j], wsem
            )

        def process(l, s):
            transpose(s, 0, 0)
            w0 = fire_write(l, 0, 0)
            transpose(s, 1, 1)
            w1 = fire_write(l, 1, 1)
            w0.wait()
            transpose(s, 2, 0)
            w2 = fire_write(l, 2, 0)
            w1.wait()
            transpose(s, 3, 1)
            w3 = fire_write(l, 3, 1)
            w2.wait()
            w3.wait()

        def wait_gather(l, s):
            pltpu.make_async_copy(tab_hbm.at[idxall.at[l]], rows.at[s], gsem).wait()

        fire_gather(0, 0)
        fire_gather(1, 1)

        @pl.loop(0, L, step=2)
        def _(g):
            wait_gather(g, 0)
            process(g, 0)

            @pl.when(g + 2 < L)
            def _():
                fire_gather(g + 2, 0)

            wait_gather(g + 1, 1)
            process(g + 1, 1)

            @pl.when(g + 3 < L)
            def _():
                fire_gather(g + 3, 1)

    out5 = gather_kernel(tablep, idx_t)
    return out5.transpose(2, 4, 0, 1, 3).reshape(B, L, D)


# speedup vs baseline: 1.5631x; 1.0151x over previous
import jax
import jax.numpy as jnp
from jax import lax
from jax.experimental import pallas as pl
from jax.experimental.pallas import tpu as pltpu
from jax.experimental.pallas import tpu_sc as plsc

B = 16384
L = 50
D = 64
N = B * L
VOCAB = 1000000

NW = 32
BB_PER_W = (B // 128) // NW
ITEMS = L * BB_PER_W


def kernel(indices, table):
    tablep = jnp.pad(table, ((0, 0), (0, 128 - D))).reshape(2 * VOCAB, D)
    idx_t = indices.T.astype(jnp.int32) * 2

    mesh = plsc.VectorSubcoreMesh(core_axis_name="core", subcore_axis_name="subcore")

    @pl.kernel(
        out_type=jax.ShapeDtypeStruct((L, 8, 128, 8, 128), table.dtype),
        mesh=mesh,
        scratch_types=[
            pltpu.VMEM((L, 128 * BB_PER_W), jnp.int32),
            pltpu.VMEM((2, 128 * BB_PER_W, D), jnp.float32),
            pltpu.VMEM((4, 8, 8, 128), jnp.float32),
            pltpu.SemaphoreType.DMA,
            pltpu.SemaphoreType.DMA,
            pltpu.SemaphoreType.DMA,
        ],
        compiler_params=pltpu.CompilerParams(
            use_tc_tiling_on_sc=False, needs_layout_passes=False
        ),
    )
    def gather_kernel(tab_hbm, idx_hbm, out_hbm, idxall, rows, tbuf, isem, gsem, wsem):
        wid = lax.axis_index("subcore") * 2 + lax.axis_index("core")
        base_b = wid * (128 * BB_PER_W)
        pltpu.async_copy(idx_hbm.at[:, pl.ds(base_b, 128 * BB_PER_W)], idxall, isem).wait()

        iota = lax.iota(jnp.int32, 16)
        ds_vec = lax.rem(iota, 8)
        db_vecs = [iota // 8 + 2 * k for k in range(4)]

        def fire_gather(l, s):
            return pltpu.async_copy(tab_hbm.at[idxall.at[l]], rows.at[s], gsem)

        def transpose(s, j, tb):
            @plsc.parallel_loop(0, 128, unroll=8)
            def _(bs):
                bs_vec = jnp.full((16,), 0, jnp.int32) + bs
                for k in range(4):
                    v = rows[s, j * 128 + bs, pl.ds(16 * k, 16)]
                    plsc.store_scatter(tbuf.at[tb], [db_vecs[k], ds_vec, bs_vec], v)

        def fire_write(l, j, tb):
            return pltpu.async_copy(
                tbuf.at[tb], out_hbm.at[l, :, wid * BB_PER_W + j], wsem
            )

        def drain_write(l, j):
            pltpu.make_async_copy(
                tbuf.at[j], out_hbm.at[l, :, wid * BB_PER_W + j], wsem
            ).wait()

        def process(l, s):
            for j in range(4):
                @pl.when(l >= 1)
                def _():
                    drain_write(l, j)

                transpose(s, j, j)
                fire_write(l, j, j)

        def wait_gather(l, s):
            pltpu.make_async_copy(tab_hbm.at[idxall.at[l]], rows.at[s], gsem).wait()

        fire_gather(0, 0)
        fire_gather(1, 1)

        @pl.loop(0, L, step=2)
        def _(g):
            wait_gather(g, 0)
            process(g, 0)

            @pl.when(g + 2 < L)
            def _():
                fire_gather(g + 2, 0)

            wait_gather(g + 1, 1)
            process(g + 1, 1)

            @pl.when(g + 3 < L)
            def _():
                fire_gather(g + 3, 1)

        for j in range(4):
            drain_write(L - 1, j)

    out5 = gather_kernel(tablep, idx_t)
    return out5.transpose(2, 4, 0, 1, 3).reshape(B, L, D)
